# baseline (speedup 1.0000x reference)
import functools

import jax
import jax.numpy as jnp
import numpy as np
from jax import lax
from jax.experimental import pallas as pl
from jax.experimental.pallas import tpu as pltpu
from jax.experimental.pallas import tpu_sc as plsc

_BETA = 0.2
_PERM_CACHE = {}


def _perm_indices(batch, ncols):
    key = (batch, ncols)
    if key not in _PERM_CACHE:
        with jax.ensure_compile_time_eval():
            perm = np.asarray(jax.random.permutation(jax.random.key(1), batch))
        _PERM_CACHE[key] = (perm.astype(np.int32) * np.int32(ncols))
    return _PERM_CACHE[key]


def _sc_gather(ind_flat, pos_flat, idx, nw, chunks, chunk):
    mesh = plsc.VectorSubcoreMesh(core_axis_name="c", subcore_axis_name="s")
    nc = 2

    @functools.partial(
        pl.kernel,
        mesh=mesh,
        out_type=[
            jax.ShapeDtypeStruct((nw, chunks, chunk), jnp.float32),
            jax.ShapeDtypeStruct((nw, chunks, chunk), jnp.float32),
        ],
        scratch_types=[
            pltpu.VMEM((chunks, chunk), jnp.int32),
            pltpu.VMEM((chunks, chunk), jnp.float32),
            pltpu.VMEM((chunks, chunk), jnp.float32),
            pltpu.SemaphoreType.DMA,
            pltpu.SemaphoreType.DMA,
        ],
    )
    def gather_kernel(ind_hbm, pos_hbm, idx_hbm, si_hbm, sp_hbm,
                      idx_v, a_v, b_v, sem_a, sem_b):
        wid = lax.axis_index("s") * nc + lax.axis_index("c")
        pltpu.sync_copy(idx_hbm.at[wid], idx_v)
        copies = []
        for j in range(chunks):
            copies.append(
                pltpu.async_copy(ind_hbm.at[idx_v.at[j]], a_v.at[j], sem_a))
            copies.append(
                pltpu.async_copy(pos_hbm.at[idx_v.at[j]], b_v.at[j], sem_b))
        for cp in copies:
            cp.wait()
        pltpu.sync_copy(a_v, si_hbm.at[wid])
        pltpu.sync_copy(b_v, sp_hbm.at[wid])

    return gather_kernel(ind_flat, pos_flat, idx)


def _tc_body(ind_ref, pos_ref, si_ref, sp_ref, out_ref):
    si = si_ref[...]
    sp = sp_ref[...]
    ind = ind_ref[...]
    pos = pos_ref[...]
    f = jnp.maximum(ind - si, 0.0)
    out_ref[...] = jnp.maximum(sp * f - pos * f + _BETA, 0.0) * f


def _tc_loss(indicator_vectors, positive, si, sp, bb):
    b, c = indicator_vectors.shape
    return pl.pallas_call(
        _tc_body,
        grid=(b // bb,),
        in_specs=[
            pl.BlockSpec((bb, c), lambda i: (i, 0)),
            pl.BlockSpec((bb, c), lambda i: (i, 0)),
            pl.BlockSpec((bb, 1), lambda i: (i, 0)),
            pl.BlockSpec((bb, 1), lambda i: (i, 0)),
        ],
        out_specs=pl.BlockSpec((bb, c), lambda i: (i, 0)),
        out_shape=jax.ShapeDtypeStruct((b, c), jnp.float32),
    )(indicator_vectors, positive, si, sp)


def kernel(indicator_vectors, positive):
    b, c = positive.shape
    nw, chunk = 32, 128
    chunks = b // (nw * chunk)
    idx = jnp.asarray(_perm_indices(b, c).reshape(nw, chunks, chunk))
    si, sp = _sc_gather(
        indicator_vectors.reshape(b * c),
        positive.reshape(b * c),
        idx, nw, chunks, chunk)
    si = si.reshape(b, 1)
    sp = sp.reshape(b, 1)
    return _tc_loss(indicator_vectors, positive, si, sp, bb=256)


# speedup vs baseline: 1.7420x; 1.7420x over previous
import functools

import jax
import jax.numpy as jnp
import numpy as np
from jax import lax
from jax.experimental import pallas as pl
from jax.experimental.pallas import tpu as pltpu
from jax.experimental.pallas import tpu_sc as plsc

_BETA = 0.2
_PERM_CACHE = {}


def _perm_indices(batch):
    key = batch
    if key not in _PERM_CACHE:
        with jax.ensure_compile_time_eval():
            perm = np.asarray(jax.random.permutation(jax.random.key(1), batch))
        _PERM_CACHE[key] = perm.astype(np.int32)
    return _PERM_CACHE[key]


def _sc_gather(ind_col, pos_col, idx, nw, chunks, chunk):
    mesh = plsc.VectorSubcoreMesh(core_axis_name="c", subcore_axis_name="s")
    nc = 2

    @functools.partial(
        pl.kernel,
        mesh=mesh,
        out_type=[
            jax.ShapeDtypeStruct((nw, chunks, chunk), jnp.float32),
            jax.ShapeDtypeStruct((nw, chunks, chunk), jnp.float32),
        ],
        scratch_types=[
            pltpu.VMEM((chunks, chunk), jnp.int32),
            pltpu.VMEM((chunks, chunk), jnp.float32),
            pltpu.VMEM((chunks, chunk), jnp.float32),
            pltpu.SemaphoreType.DMA,
            pltpu.SemaphoreType.DMA,
        ],
    )
    def gather_kernel(ind_hbm, pos_hbm, idx_hbm, si_hbm, sp_hbm,
                      idx_v, a_v, b_v, sem_a, sem_b):
        wid = lax.axis_index("s") * nc + lax.axis_index("c")
        pltpu.sync_copy(idx_hbm.at[wid], idx_v)
        copies = []
        for j in range(chunks):
            copies.append(
                pltpu.async_copy(ind_hbm.at[idx_v.at[j]], a_v.at[j], sem_a))
            copies.append(
                pltpu.async_copy(pos_hbm.at[idx_v.at[j]], b_v.at[j], sem_b))
        for cp in copies:
            cp.wait()
        pltpu.sync_copy(a_v, si_hbm.at[wid])
        pltpu.sync_copy(b_v, sp_hbm.at[wid])

    return gather_kernel(ind_col, pos_col, idx)


def _tc_body(ind_ref, pos_ref, si_ref, sp_ref, out_ref):
    si = si_ref[...]
    sp = sp_ref[...]
    ind = ind_ref[...]
    pos = pos_ref[...]
    f = jnp.maximum(ind - si, 0.0)
    out_ref[...] = jnp.maximum(sp * f - pos * f + _BETA, 0.0) * f


def _tc_loss(indicator_vectors, positive, si, sp, bb):
    b, c = indicator_vectors.shape
    return pl.pallas_call(
        _tc_body,
        grid=(b // bb,),
        in_specs=[
            pl.BlockSpec((bb, c), lambda i: (i, 0)),
            pl.BlockSpec((bb, c), lambda i: (i, 0)),
            pl.BlockSpec((bb, 1), lambda i: (i, 0)),
            pl.BlockSpec((bb, 1), lambda i: (i, 0)),
        ],
        out_specs=pl.BlockSpec((bb, c), lambda i: (i, 0)),
        out_shape=jax.ShapeDtypeStruct((b, c), jnp.float32),
    )(indicator_vectors, positive, si, sp)


def kernel(indicator_vectors, positive):
    b, c = positive.shape
    nw, chunk = 32, 128
    chunks = b // (nw * chunk)
    idx = jnp.asarray(_perm_indices(b).reshape(nw, chunks, chunk))
    si, sp = _sc_gather(
        indicator_vectors[:, 0], positive[:, 0], idx, nw, chunks, chunk)
    si = si.reshape(b, 1)
    sp = sp.reshape(b, 1)
    return _tc_loss(indicator_vectors, positive, si, sp, bb=256)


# speedup vs baseline: 2.0124x; 1.1553x over previous
import functools

import jax
import jax.numpy as jnp
import numpy as np
from jax import lax
from jax.experimental import pallas as pl
from jax.experimental.pallas import tpu as pltpu
from jax.experimental.pallas import tpu_sc as plsc

_BETA = 0.2
_PERM_CACHE = {}


def _perm_indices(batch):
    key = batch
    if key not in _PERM_CACHE:
        with jax.ensure_compile_time_eval():
            perm = np.asarray(jax.random.permutation(jax.random.key(1), batch))
        _PERM_CACHE[key] = perm.astype(np.int32)
    return _PERM_CACHE[key]


def _sc_gather(ind_col, pos_col, idx, nw, chunks, chunk):
    mesh = plsc.VectorSubcoreMesh(core_axis_name="c", subcore_axis_name="s")
    nc = 2

    @functools.partial(
        pl.kernel,
        mesh=mesh,
        out_type=[
            jax.ShapeDtypeStruct((nw, chunks, chunk), jnp.float32),
            jax.ShapeDtypeStruct((nw, chunks, chunk), jnp.float32),
        ],
        scratch_types=[
            pltpu.VMEM((chunks, chunk), jnp.int32),
            pltpu.VMEM((chunks, chunk), jnp.float32),
            pltpu.VMEM((chunks, chunk), jnp.float32),
            pltpu.SemaphoreType.DMA,
            pltpu.SemaphoreType.DMA,
        ],
    )
    def gather_kernel(ind_hbm, pos_hbm, idx_hbm, si_hbm, sp_hbm,
                      idx_v, a_v, b_v, sem_a, sem_b):
        wid = lax.axis_index("s") * nc + lax.axis_index("c")
        pltpu.sync_copy(idx_hbm.at[wid], idx_v)
        copies = []
        for j in range(chunks):
            copies.append(
                pltpu.async_copy(ind_hbm.at[idx_v.at[j]], a_v.at[j], sem_a))
            copies.append(
                pltpu.async_copy(pos_hbm.at[idx_v.at[j]], b_v.at[j], sem_b))
        for cp in copies:
            cp.wait()
        pltpu.sync_copy(a_v, si_hbm.at[wid])
        pltpu.sync_copy(b_v, sp_hbm.at[wid])

    return gather_kernel(ind_col, pos_col, idx)


def _tc_body(ind_ref, pos_ref, si_ref, sp_ref, out_ref):
    si = si_ref[...]
    sp = sp_ref[...]
    ind = ind_ref[...]
    pos = pos_ref[...]
    f = jnp.maximum(ind - si, 0.0)
    out_ref[...] = jnp.maximum(sp * f - pos * f + _BETA, 0.0) * f


def _tc_loss(indicator_vectors, positive, si, sp, bb):
    b, c = indicator_vectors.shape
    return pl.pallas_call(
        _tc_body,
        grid=(b // bb,),
        in_specs=[
            pl.BlockSpec((bb, c), lambda i: (i, 0)),
            pl.BlockSpec((bb, c), lambda i: (i, 0)),
            pl.BlockSpec((bb, 1), lambda i: (i, 0)),
            pl.BlockSpec((bb, 1), lambda i: (i, 0)),
        ],
        out_specs=pl.BlockSpec((bb, c), lambda i: (i, 0)),
        out_shape=jax.ShapeDtypeStruct((b, c), jnp.float32),
    )(indicator_vectors, positive, si, sp)


def kernel(indicator_vectors, positive):
    b, c = positive.shape
    nw, chunk = 32, 128
    chunks = b // (nw * chunk)
    idx = jnp.asarray(_perm_indices(b).reshape(nw, chunks, chunk))
    si, sp = _sc_gather(
        indicator_vectors[:, 0], positive[:, 0], idx, nw, chunks, chunk)
    si = si.reshape(b, 1)
    sp = sp.reshape(b, 1)
    return _tc_loss(indicator_vectors, positive, si, sp, bb=1024)
